# baseline (speedup 1.0000x reference)
import jax
import jax.numpy as jnp
from jax.experimental import pallas as pl


def kernel(tensor, batch_num_nodes, weight, bias, mean_scale):
    n, c = tensor.shape
    b = batch_num_nodes.shape[0]
    seg = n // b
    inv = 1.0 / seg

    def body(x_ref, w_ref, b_ref, ms_ref, o_ref):
        x = x_ref[...]
        m = jnp.sum(x, axis=0, keepdims=True) * inv
        a = m * ms_ref[...]
        sub = x - a
        var = jnp.sum(sub * sub, axis=0, keepdims=True) * inv
        o_ref[...] = w_ref[...] * sub * jax.lax.rsqrt(var + 1e-6) + b_ref[...]

    return pl.pallas_call(
        body,
        grid=(b,),
        in_specs=[
            pl.BlockSpec((seg, c), lambda i: (i, 0)),
            pl.BlockSpec((1, c), lambda i: (0, 0)),
            pl.BlockSpec((1, c), lambda i: (0, 0)),
            pl.BlockSpec((1, c), lambda i: (0, 0)),
        ],
        out_specs=pl.BlockSpec((seg, c), lambda i: (i, 0)),
        out_shape=jax.ShapeDtypeStruct((n, c), jnp.float32),
    )(tensor, weight.reshape(1, c), bias.reshape(1, c),
      mean_scale.reshape(1, c))
